# baseline (speedup 1.0000x reference)
import functools

import jax
import jax.numpy as jnp
from jax import lax
from jax.experimental import pallas as pl
from jax.experimental.pallas import tpu as pltpu
from jax.experimental.pallas import tpu_sc as plsc

N = 10000
E = 320000
D = 128
NC, NS = 2, 16
NW = NC * NS
EPW = E // NW
K = 80
CH = EPW // K
NPAD = 10240
STRIP = NPAD // NS
ZROWS = NPAD // NS
ZCH = 128

_mesh = plsc.VectorSubcoreMesh(core_axis_name="c", subcore_axis_name="s",
                               num_cores=NC, num_subcores=NS)


def _z16():
    return jnp.zeros((16,), jnp.float32)


def _o16():
    return jnp.ones((16,), jnp.float32)


@functools.partial(
    pl.kernel,
    out_type=jax.ShapeDtypeStruct((NC, NPAD), jnp.float32),
    mesh=_mesh,
    scratch_types=[
        pltpu.VMEM((CH, K), jnp.int32),
        pltpu.VMEM((K,), jnp.float32),
        pltpu.VMEM((STRIP,), jnp.float32),
        pltpu.VMEM_SHARED((NPAD,), jnp.float32),
    ],
)
def _deg_kernel(dst_hbm, out_hbm, idx_v, ones_v, zbuf_v, deg_sp):
    c = lax.axis_index("c")
    s = lax.axis_index("s")
    wid = s * NC + c

    def zfill(i, carry):
        zbuf_v[pl.ds(i * 16, 16)] = _z16()
        return carry

    lax.fori_loop(0, STRIP // 16, zfill, 0)

    def ofill(i, carry):
        ones_v[pl.ds(i * 16, 16)] = _o16()
        return carry

    lax.fori_loop(0, K // 16, ofill, 0)

    pltpu.sync_copy(zbuf_v, deg_sp.at[pl.ds(s * STRIP, STRIP)])
    plsc.subcore_barrier()

    pltpu.sync_copy(dst_hbm.at[wid], idx_v)

    def body(j, carry):
        pltpu.sync_copy(ones_v, deg_sp.at[idx_v.at[j]], add=True)
        return carry

    lax.fori_loop(0, CH, body, 0)
    plsc.subcore_barrier()

    pltpu.sync_copy(
        deg_sp.at[pl.ds(s * STRIP, STRIP)],
        out_hbm.at[c, pl.ds(s * STRIP, STRIP)],
    )


_B2 = 1000


def _scale_body(degp_ref, x_ref, dinv_ref, xpa_ref, xpb_ref):
    deg = degp_ref[0] + degp_ref[1] + 1.0
    dinv = lax.rsqrt(deg)
    dinv_ref[...] = dinv
    xpa_ref[...] = x_ref[:, : D // 2] * dinv
    xpb_ref[...] = x_ref[:, D // 2:] * dinv


def _scale_call(deg_p3, x):
    return pl.pallas_call(
        _scale_body,
        grid=(N // _B2,),
        in_specs=[
            pl.BlockSpec((NC, _B2, 1), lambda i: (0, i, 0)),
            pl.BlockSpec((_B2, D), lambda i: (i, 0)),
        ],
        out_specs=[
            pl.BlockSpec((_B2, 1), lambda i: (i, 0)),
            pl.BlockSpec((_B2, D // 2), lambda i: (i, 0)),
            pl.BlockSpec((_B2, D // 2), lambda i: (i, 0)),
        ],
        out_shape=[
            jax.ShapeDtypeStruct((NPAD, 1), jnp.float32),
            jax.ShapeDtypeStruct((N, D // 2), jnp.float32),
            jax.ShapeDtypeStruct((N, D // 2), jnp.float32),
        ],
    )(deg_p3, x)


DH = D // 2
EPT = E // NS
CH2 = EPT // K


NBUF = 5
NGRP = CH2 // NBUF


def _edge_loop(xp_hbm, src_v, dst_v, rows_v, z_sp, c,
               dval_v, dinv_sp, t_sp, sem_g, sem_s, sem_dg, sem_ts):
    def group(g, carry):
        base = g * NBUF
        for b in range(NBUF):
            j = base + b
            pltpu.async_copy(xp_hbm.at[src_v.at[j]], rows_v.at[b], sem_g)

            @pl.when((j % 2) == c)
            def _():
                pltpu.async_copy(dinv_sp.at[dst_v.at[j]], dval_v.at[b], sem_dg)

        for b in range(NBUF):
            j = base + b
            pltpu.make_async_copy(
                xp_hbm.at[src_v.at[j]], rows_v.at[b], sem_g).wait()
            pltpu.async_copy(rows_v.at[b], z_sp.at[dst_v.at[j]], sem_s,
                             add=True)

            @pl.when((j % 2) == c)
            def _():
                pltpu.make_async_copy(
                    dinv_sp.at[dst_v.at[j]], dval_v.at[b], sem_dg).wait()
                pltpu.async_copy(dval_v.at[b], t_sp.at[src_v.at[j]], sem_ts,
                                 add=True)

        for b in range(NBUF):
            j = base + b
            pltpu.make_async_copy(
                rows_v.at[b], z_sp.at[dst_v.at[j]], sem_s).wait()

            @pl.when((j % 2) == c)
            def _():
                pltpu.make_async_copy(
                    dval_v.at[b], t_sp.at[src_v.at[j]], sem_ts).wait()

        return carry

    lax.fori_loop(0, NGRP, group, 0)


@functools.partial(
    pl.kernel,
    out_type=(
        jax.ShapeDtypeStruct((NPAD, DH), jnp.float32),
        jax.ShapeDtypeStruct((NPAD, DH), jnp.float32),
        jax.ShapeDtypeStruct((NC, NPAD), jnp.float32),
    ),
    mesh=_mesh,
    compiler_params=pltpu.CompilerParams(use_tc_tiling_on_sc=False),
    scratch_types=[
        pltpu.VMEM((CH2, K), jnp.int32),
        pltpu.VMEM((CH2, K), jnp.int32),
        pltpu.VMEM((NBUF, K, DH), jnp.float32),
        pltpu.VMEM((NBUF, K), jnp.float32),
        pltpu.VMEM((ZCH, DH), jnp.float32),
        pltpu.VMEM((STRIP,), jnp.float32),
        pltpu.VMEM_SHARED((NPAD, DH), jnp.float32),
        pltpu.VMEM_SHARED((NPAD,), jnp.float32),
        pltpu.VMEM_SHARED((NPAD,), jnp.float32),
        pltpu.SemaphoreType.DMA,
        pltpu.SemaphoreType.DMA,
        pltpu.SemaphoreType.DMA,
        pltpu.SemaphoreType.DMA,
    ],
)
def _edge_kernel(xpa_hbm, xpb_hbm, src_hbm, dst_hbm, dinv_hbm,
                 za_hbm, zb_hbm, t_hbm,
                 src_v, dst_v, rows_v, dval_v, z2buf_v, zbuf_v,
                 z_sp, t_sp, dinv_sp, sem_g, sem_s, sem_dg, sem_ts):
    c = lax.axis_index("c")
    s = lax.axis_index("s")

    def zrow(i, carry):
        def zcol(j, carry2):
            z2buf_v[i, pl.ds(j * 16, 16)] = _z16()
            return carry2
        return lax.fori_loop(0, DH // 16, zcol, carry)

    lax.fori_loop(0, ZCH, zrow, 0)

    def zfill(i, carry):
        zbuf_v[pl.ds(i * 16, 16)] = _z16()
        return carry

    lax.fori_loop(0, STRIP // 16, zfill, 0)

    for q in range(ZROWS // ZCH):
        pltpu.sync_copy(z2buf_v, z_sp.at[pl.ds(s * ZROWS + q * ZCH, ZCH)])
    pltpu.sync_copy(zbuf_v, t_sp.at[pl.ds(s * STRIP, STRIP)])

    @pl.when(s == 0)
    def _():
        pltpu.sync_copy(dinv_hbm, dinv_sp)

    plsc.subcore_barrier()

    pltpu.sync_copy(src_hbm.at[s], src_v)
    pltpu.sync_copy(dst_hbm.at[s], dst_v)

    @pl.when(c == 0)
    def _():
        _edge_loop(xpa_hbm, src_v, dst_v, rows_v, z_sp, c,
                   dval_v, dinv_sp, t_sp, sem_g, sem_s, sem_dg, sem_ts)

    @pl.when(c == 1)
    def _():
        _edge_loop(xpb_hbm, src_v, dst_v, rows_v, z_sp, c,
                   dval_v, dinv_sp, t_sp, sem_g, sem_s, sem_dg, sem_ts)

    plsc.subcore_barrier()

    strip_rows = pl.ds(s * ZROWS, ZROWS)

    @pl.when(c == 0)
    def _():
        pltpu.sync_copy(z_sp.at[strip_rows], za_hbm.at[strip_rows])

    @pl.when(c == 1)
    def _():
        pltpu.sync_copy(z_sp.at[strip_rows], zb_hbm.at[strip_rows])

    pltpu.sync_copy(t_sp.at[pl.ds(s * STRIP, STRIP)],
                    t_hbm.at[c, pl.ds(s * STRIP, STRIP)])


_B4 = 1000
_NB4 = N // _B4


def _final_body(za_ref, zb_ref, xpa_ref, xpb_ref, dinv_ref, tp_ref,
                w1_ref, b1_ref, w2_ref, b2_ref, out_ref):
    i = pl.program_id(0)
    dinv = dinv_ref[...]
    a = jnp.concatenate(
        [za_ref[...] + xpa_ref[...], zb_ref[...] + xpb_ref[...]], axis=1
    ) * dinv
    r = jnp.dot(a, w1_ref[...], preferred_element_type=jnp.float32)
    r = jnp.maximum(r + b1_ref[...], 0.0)
    w = dinv * (tp_ref[0] + tp_ref[1] + dinv)
    part = jnp.sum(w * r, axis=0, keepdims=True)

    @pl.when(i == 0)
    def _():
        out_ref[...] = jnp.zeros_like(out_ref)

    out_ref[...] += part

    @pl.when(i == _NB4 - 1)
    def _():
        q = out_ref[...]
        out_ref[...] = (
            jnp.dot(q, w2_ref[...], preferred_element_type=jnp.float32) / N
            + b2_ref[...]
        )


def _final_call(za, zb, xpa, xpb, dinv, tp, W1, b1, W2, b2):
    blk = pl.BlockSpec((_B4, DH), lambda i: (i, 0))
    col = pl.BlockSpec((_B4, 1), lambda i: (i, 0))
    tcol = pl.BlockSpec((NC, _B4, 1), lambda i: (0, i, 0))
    mat = pl.BlockSpec((D, D), lambda i: (0, 0))
    row = pl.BlockSpec((1, D), lambda i: (0, 0))
    return pl.pallas_call(
        _final_body,
        grid=(_NB4,),
        in_specs=[blk, blk, blk, blk, col, tcol, mat, row, mat, row],
        out_specs=pl.BlockSpec((1, D), lambda i: (0, 0)),
        out_shape=jax.ShapeDtypeStruct((1, D), jnp.float32),
    )(za, zb, xpa, xpb, dinv, tp, W1, b1, W2, b2)


def kernel(x, edge_index, W1, b1, W2, b2):
    dst3 = edge_index[1].reshape(NW, CH, K)
    src2 = edge_index[0].reshape(NS, CH2, K)
    dst2 = edge_index[1].reshape(NS, CH2, K)

    deg_p = _deg_kernel(dst3)
    dinv, xpa, xpb = _scale_call(deg_p.reshape(NC, NPAD, 1), x)

    za, zb, t_p = _edge_kernel(xpa, xpb, src2, dst2, dinv.reshape(NPAD))

    return _final_call(za, zb, xpa, xpb, dinv,
                       t_p.reshape(NC, NPAD, 1),
                       W1, b1.reshape(1, D), W2, b2.reshape(1, D))


# speedup vs baseline: 1.0843x; 1.0843x over previous
import functools

import jax
import jax.numpy as jnp
from jax import lax
from jax.experimental import pallas as pl
from jax.experimental.pallas import tpu as pltpu
from jax.experimental.pallas import tpu_sc as plsc

N = 10000
E = 320000
D = 128
NC, NS = 2, 16
NW = NC * NS
EPW = E // NW
K = 80
CH = EPW // K
NPAD = 10240
STRIP = NPAD // NS
ZROWS = NPAD // NS
ZCH = 128

_mesh = plsc.VectorSubcoreMesh(core_axis_name="c", subcore_axis_name="s",
                               num_cores=NC, num_subcores=NS)


def _z16():
    return jnp.zeros((16,), jnp.float32)


def _o16():
    return jnp.ones((16,), jnp.float32)


@functools.partial(
    pl.kernel,
    out_type=jax.ShapeDtypeStruct((NC, NPAD), jnp.float32),
    mesh=_mesh,
    scratch_types=[
        pltpu.VMEM((CH, K), jnp.int32),
        pltpu.VMEM((K,), jnp.float32),
        pltpu.VMEM((STRIP,), jnp.float32),
        pltpu.VMEM_SHARED((NPAD,), jnp.float32),
    ],
)
def _deg_kernel(dst_hbm, out_hbm, idx_v, ones_v, zbuf_v, deg_sp):
    c = lax.axis_index("c")
    s = lax.axis_index("s")
    wid = s * NC + c

    def zfill(i, carry):
        zbuf_v[pl.ds(i * 16, 16)] = _z16()
        return carry

    lax.fori_loop(0, STRIP // 16, zfill, 0)

    def ofill(i, carry):
        ones_v[pl.ds(i * 16, 16)] = _o16()
        return carry

    lax.fori_loop(0, K // 16, ofill, 0)

    pltpu.sync_copy(zbuf_v, deg_sp.at[pl.ds(s * STRIP, STRIP)])
    plsc.subcore_barrier()

    pltpu.sync_copy(dst_hbm.at[wid], idx_v)

    def body(j, carry):
        pltpu.sync_copy(ones_v, deg_sp.at[idx_v.at[j]], add=True)
        return carry

    lax.fori_loop(0, CH, body, 0)
    plsc.subcore_barrier()

    pltpu.sync_copy(
        deg_sp.at[pl.ds(s * STRIP, STRIP)],
        out_hbm.at[c, pl.ds(s * STRIP, STRIP)],
    )


_B2 = 1000


def _scale_body(degp_ref, x_ref, dinv_ref, xpa_ref, xpb_ref):
    deg = degp_ref[0] + degp_ref[1] + 1.0
    dinv = lax.rsqrt(deg)
    dinv_ref[...] = dinv
    xpa_ref[...] = x_ref[:, : D // 2] * dinv
    xpb_ref[...] = x_ref[:, D // 2:] * dinv


def _scale_call(deg_p3, x):
    return pl.pallas_call(
        _scale_body,
        grid=(N // _B2,),
        in_specs=[
            pl.BlockSpec((NC, _B2, 1), lambda i: (0, i, 0)),
            pl.BlockSpec((_B2, D), lambda i: (i, 0)),
        ],
        out_specs=[
            pl.BlockSpec((_B2, 1), lambda i: (i, 0)),
            pl.BlockSpec((_B2, D // 2), lambda i: (i, 0)),
            pl.BlockSpec((_B2, D // 2), lambda i: (i, 0)),
        ],
        out_shape=[
            jax.ShapeDtypeStruct((NPAD, 1), jnp.float32),
            jax.ShapeDtypeStruct((N, D // 2), jnp.float32),
            jax.ShapeDtypeStruct((N, D // 2), jnp.float32),
        ],
    )(deg_p3, x)


DH = D // 2
EPT = E // NS
CH2 = EPT // K


NBUF = 10
NGRP = CH2 // NBUF


def _edge_loop(xp_hbm, src_hbm, dst_hbm, s, c, srcg_v, dstg_v, rows_v,
               dval_v, z_sp, dinv_sp, t_sp,
               sem_g, sem_s, sem_dg, sem_ts, sem_i):
    pltpu.sync_copy(src_hbm.at[s, pl.ds(0, NBUF)], srcg_v.at[0])
    pltpu.sync_copy(dst_hbm.at[s, pl.ds(0, NBUF)], dstg_v.at[0])

    def group(g, carry):
        slot = g % 2

        for b in range(NBUF):
            @pl.when(g > 0)
            def _():
                pltpu.make_async_copy(
                    rows_v.at[b], z_sp.at[dstg_v.at[slot, b]], sem_s).wait()

            @pl.when(((b % 2) == c) & (g > 0))
            def _():
                pltpu.make_async_copy(
                    dval_v.at[b], t_sp.at[srcg_v.at[slot, b]], sem_ts).wait()

            pltpu.async_copy(xp_hbm.at[srcg_v.at[slot, b]], rows_v.at[b],
                             sem_g)

            @pl.when((b % 2) == c)
            def _():
                pltpu.async_copy(dinv_sp.at[dstg_v.at[slot, b]],
                                 dval_v.at[b], sem_dg)

        @pl.when(g + 1 < NGRP)
        def _():
            nxt = pl.ds((g + 1) * NBUF, NBUF)
            pltpu.async_copy(src_hbm.at[s, nxt], srcg_v.at[1 - slot], sem_i)
            pltpu.async_copy(dst_hbm.at[s, nxt], dstg_v.at[1 - slot], sem_i)

        for b in range(NBUF):
            pltpu.make_async_copy(
                xp_hbm.at[srcg_v.at[slot, b]], rows_v.at[b], sem_g).wait()
            pltpu.async_copy(rows_v.at[b], z_sp.at[dstg_v.at[slot, b]],
                             sem_s, add=True)

            @pl.when((b % 2) == c)
            def _():
                pltpu.make_async_copy(
                    dinv_sp.at[dstg_v.at[slot, b]], dval_v.at[b],
                    sem_dg).wait()
                pltpu.async_copy(dval_v.at[b], t_sp.at[srcg_v.at[slot, b]],
                                 sem_ts, add=True)

        @pl.when(g + 1 < NGRP)
        def _():
            pltpu.make_async_copy(
                src_hbm.at[s, pl.ds(0, NBUF)], srcg_v.at[0], sem_i).wait()
            pltpu.make_async_copy(
                dst_hbm.at[s, pl.ds(0, NBUF)], dstg_v.at[0], sem_i).wait()

        return carry

    lax.fori_loop(0, NGRP, group, 0)

    for b in range(NBUF):
        pltpu.make_async_copy(
            rows_v.at[b], z_sp.at[dstg_v.at[0, b]], sem_s).wait()

        @pl.when((b % 2) == c)
        def _():
            pltpu.make_async_copy(
                dval_v.at[b], t_sp.at[srcg_v.at[0, b]], sem_ts).wait()


@functools.partial(
    pl.kernel,
    out_type=(
        jax.ShapeDtypeStruct((NPAD, DH), jnp.float32),
        jax.ShapeDtypeStruct((NPAD, DH), jnp.float32),
        jax.ShapeDtypeStruct((NC, NPAD), jnp.float32),
    ),
    mesh=_mesh,
    compiler_params=pltpu.CompilerParams(use_tc_tiling_on_sc=False),
    scratch_types=[
        pltpu.VMEM((2, NBUF, K), jnp.int32),
        pltpu.VMEM((2, NBUF, K), jnp.int32),
        pltpu.VMEM((NBUF, K, DH), jnp.float32),
        pltpu.VMEM((NBUF, K), jnp.float32),
        pltpu.VMEM((ZCH, DH), jnp.float32),
        pltpu.VMEM((STRIP,), jnp.float32),
        pltpu.VMEM_SHARED((NPAD, DH), jnp.float32),
        pltpu.VMEM_SHARED((NPAD,), jnp.float32),
        pltpu.VMEM_SHARED((NPAD,), jnp.float32),
        pltpu.SemaphoreType.DMA,
        pltpu.SemaphoreType.DMA,
        pltpu.SemaphoreType.DMA,
        pltpu.SemaphoreType.DMA,
        pltpu.SemaphoreType.DMA,
    ],
)
def _edge_kernel(xpa_hbm, xpb_hbm, src_hbm, dst_hbm, dinv_hbm,
                 za_hbm, zb_hbm, t_hbm,
                 srcg_v, dstg_v, rows_v, dval_v, z2buf_v, zbuf_v,
                 z_sp, t_sp, dinv_sp, sem_g, sem_s, sem_dg, sem_ts, sem_i):
    c = lax.axis_index("c")
    s = lax.axis_index("s")

    def zrow(i, carry):
        def zcol(j, carry2):
            z2buf_v[i, pl.ds(j * 16, 16)] = _z16()
            return carry2
        return lax.fori_loop(0, DH // 16, zcol, carry)

    lax.fori_loop(0, ZCH, zrow, 0)

    def zfill(i, carry):
        zbuf_v[pl.ds(i * 16, 16)] = _z16()
        return carry

    lax.fori_loop(0, STRIP // 16, zfill, 0)

    for q in range(ZROWS // ZCH):
        pltpu.sync_copy(z2buf_v, z_sp.at[pl.ds(s * ZROWS + q * ZCH, ZCH)])
    pltpu.sync_copy(zbuf_v, t_sp.at[pl.ds(s * STRIP, STRIP)])

    @pl.when(s == 0)
    def _():
        pltpu.sync_copy(dinv_hbm, dinv_sp)

    plsc.subcore_barrier()

    @pl.when(c == 0)
    def _():
        _edge_loop(xpa_hbm, src_hbm, dst_hbm, s, c, srcg_v, dstg_v, rows_v,
                   dval_v, z_sp, dinv_sp, t_sp,
                   sem_g, sem_s, sem_dg, sem_ts, sem_i)

    @pl.when(c == 1)
    def _():
        _edge_loop(xpb_hbm, src_hbm, dst_hbm, s, c, srcg_v, dstg_v, rows_v,
                   dval_v, z_sp, dinv_sp, t_sp,
                   sem_g, sem_s, sem_dg, sem_ts, sem_i)

    plsc.subcore_barrier()

    strip_rows = pl.ds(s * ZROWS, ZROWS)

    @pl.when(c == 0)
    def _():
        pltpu.sync_copy(z_sp.at[strip_rows], za_hbm.at[strip_rows])

    @pl.when(c == 1)
    def _():
        pltpu.sync_copy(z_sp.at[strip_rows], zb_hbm.at[strip_rows])

    pltpu.sync_copy(t_sp.at[pl.ds(s * STRIP, STRIP)],
                    t_hbm.at[c, pl.ds(s * STRIP, STRIP)])


_B4 = 1000
_NB4 = N // _B4


def _final_body(za_ref, zb_ref, xpa_ref, xpb_ref, dinv_ref, tp_ref,
                w1_ref, b1_ref, w2_ref, b2_ref, out_ref):
    i = pl.program_id(0)
    dinv = dinv_ref[...]
    a = jnp.concatenate(
        [za_ref[...] + xpa_ref[...], zb_ref[...] + xpb_ref[...]], axis=1
    ) * dinv
    r = jnp.dot(a, w1_ref[...], preferred_element_type=jnp.float32)
    r = jnp.maximum(r + b1_ref[...], 0.0)
    w = dinv * (tp_ref[0] + tp_ref[1] + dinv)
    part = jnp.sum(w * r, axis=0, keepdims=True)

    @pl.when(i == 0)
    def _():
        out_ref[...] = jnp.zeros_like(out_ref)

    out_ref[...] += part

    @pl.when(i == _NB4 - 1)
    def _():
        q = out_ref[...]
        out_ref[...] = (
            jnp.dot(q, w2_ref[...], preferred_element_type=jnp.float32) / N
            + b2_ref[...]
        )


def _final_call(za, zb, xpa, xpb, dinv, tp, W1, b1, W2, b2):
    blk = pl.BlockSpec((_B4, DH), lambda i: (i, 0))
    col = pl.BlockSpec((_B4, 1), lambda i: (i, 0))
    tcol = pl.BlockSpec((NC, _B4, 1), lambda i: (0, i, 0))
    mat = pl.BlockSpec((D, D), lambda i: (0, 0))
    row = pl.BlockSpec((1, D), lambda i: (0, 0))
    return pl.pallas_call(
        _final_body,
        grid=(_NB4,),
        in_specs=[blk, blk, blk, blk, col, tcol, mat, row, mat, row],
        out_specs=pl.BlockSpec((1, D), lambda i: (0, 0)),
        out_shape=jax.ShapeDtypeStruct((1, D), jnp.float32),
    )(za, zb, xpa, xpb, dinv, tp, W1, b1, W2, b2)


def kernel(x, edge_index, W1, b1, W2, b2):
    dst3 = edge_index[1].reshape(NW, CH, K)
    src2 = edge_index[0].reshape(NS, CH2, K)
    dst2 = edge_index[1].reshape(NS, CH2, K)

    deg_p = _deg_kernel(dst3)
    dinv, xpa, xpb = _scale_call(deg_p.reshape(NC, NPAD, 1), x)

    za, zb, t_p = _edge_kernel(xpa, xpb, src2, dst2, dinv.reshape(NPAD))

    return _final_call(za, zb, xpa, xpb, dinv,
                       t_p.reshape(NC, NPAD, 1),
                       W1, b1.reshape(1, D), W2, b2.reshape(1, D))
